# baseline (speedup 1.0000x reference)
import functools

import jax
import jax.numpy as jnp
from jax import lax
from jax.experimental import pallas as pl
from jax.experimental.pallas import tpu as pltpu
from jax.experimental.pallas import tpu_sc as plsc

EPS = 0.001

_NC = 2
_NS = 16
_L = 16


def _fold_params_kernel(g_ref, b_ref, m_ref, v_ref, p_ref, a_out, b_out):
    inv = jax.lax.rsqrt(v_ref[...] + EPS) * g_ref[...]
    rp = jax.lax.rsqrt(p_ref[...])
    a_out[...] = inv * rp
    b_out[...] = (b_ref[...] - m_ref[...] * inv) * rp


def _make_sc_apply(N, D, C, rows_per_chunk):
    nw = _NC * _NS
    rows_per_w = N // nw
    n_chunks = rows_per_w // rows_per_chunk
    mesh = plsc.VectorSubcoreMesh(core_axis_name="c", subcore_axis_name="s")

    @functools.partial(
        pl.kernel,
        mesh=mesh,
        out_type=jax.ShapeDtypeStruct((N, D), jnp.float32),
        scratch_types=[
            pltpu.VMEM((rows_per_chunk, D), jnp.float32),
            pltpu.VMEM((rows_per_chunk, D), jnp.float32),
            pltpu.VMEM((rows_per_chunk, D), jnp.float32),
            pltpu.VMEM((rows_per_chunk, D), jnp.float32),
            pltpu.VMEM((rows_per_chunk, D), jnp.float32),
            pltpu.VMEM((rows_per_chunk, D), jnp.float32),
            pltpu.VMEM((rows_per_w,), jnp.int32),
            pltpu.VMEM((C * D,), jnp.uint32),
            pltpu.SemaphoreType.DMA,
            pltpu.SemaphoreType.DMA,
            pltpu.SemaphoreType.DMA,
            pltpu.SemaphoreType.DMA,
            pltpu.SemaphoreType.DMA,
            pltpu.SemaphoreType.DMA,
        ],
    )
    def sc_apply(x_hbm, ctx_hbm, ab_hbm, out_hbm,
                 xbuf0, xbuf1, xbuf2, obuf0, obuf1, obuf2, ctxbuf, abbuf,
                 isem0, isem1, isem2, osem0, osem1, osem2):
        wid = lax.axis_index("s") * _NC + lax.axis_index("c")
        base = wid * rows_per_w
        pltpu.sync_copy(ab_hbm, abbuf)
        pltpu.sync_copy(ctx_hbm.at[pl.ds(base, rows_per_w)], ctxbuf)

        xbufs, obufs = (xbuf0, xbuf1, xbuf2), (obuf0, obuf1, obuf2)
        isems, osems = (isem0, isem1, isem2), (osem0, osem1, osem2)
        unroll = 2
        depth = 3
        n_groups = n_chunks // depth

        def start_in(i, slot):
            rb = base + i * rows_per_chunk
            pltpu.async_copy(x_hbm.at[pl.ds(rb, rows_per_chunk)],
                             xbufs[slot], isems[slot])

        def wait_in(i, slot):
            rb = base + i * rows_per_chunk
            pltpu.make_async_copy(x_hbm.at[pl.ds(rb, rows_per_chunk)],
                                  xbufs[slot], isems[slot]).wait()

        def start_out(i, slot):
            rb = base + i * rows_per_chunk
            pltpu.async_copy(obufs[slot],
                             out_hbm.at[pl.ds(rb, rows_per_chunk)], osems[slot])

        def wait_out(i, slot):
            rb = base + i * rows_per_chunk
            pltpu.make_async_copy(obufs[slot],
                                  out_hbm.at[pl.ds(rb, rows_per_chunk)],
                                  osems[slot]).wait()

        def compute(i, slot):
            xbuf, obuf = xbufs[slot], obufs[slot]
            cvec = ctxbuf[pl.ds(i * rows_per_chunk, _L)]
            cbase = cvec * D
            cbs = [cbase[r] for r in range(rows_per_chunk)]

            @plsc.parallel_loop(0, D, _L, unroll=unroll)
            def col_body(off):
                sl = pl.ds(off, _L)
                for r in range(rows_per_chunk):
                    w = abbuf[pl.ds(cbs[r] + off, _L)]
                    a = jax.lax.bitcast_convert_type(w << 16, jnp.float32)
                    b = jax.lax.bitcast_convert_type(
                        w & jnp.uint32(0xFFFF0000), jnp.float32)
                    obuf[r, sl] = xbuf[r, sl] * a + b

        for s in range(depth):
            start_in(s, s)

        def group_body(g, _):
            for s in range(depth):
                i = depth * g + s
                wait_in(i, s)

                @pl.when(g > 0)
                def _():
                    wait_out(i - depth, s)

                compute(i, s)
                start_out(i, s)

                @pl.when(i + depth < n_chunks)
                def _():
                    start_in(i + depth, s)

            return 0

        lax.fori_loop(0, n_groups, group_body, 0)
        for i in range(depth * n_groups, n_chunks):
            s = i % depth
            wait_in(i, s)
            wait_out(i - depth, s)
            compute(i, s)
            start_out(i, s)
        for i in range(n_chunks - depth, n_chunks):
            wait_out(i, i % depth)

    return sc_apply


def kernel(samples, contexts, gamma, beta, moving_mean, moving_var, priors):
    N, D = samples.shape
    C = gamma.shape[0]
    a_tab, b_tab = pl.pallas_call(
        _fold_params_kernel,
        out_shape=(jax.ShapeDtypeStruct((C, D), jnp.float32),
                   jax.ShapeDtypeStruct((C, D), jnp.float32)),
    )(gamma, beta, moving_mean, moving_var, priors.reshape(C, 1))

    a_bits = jax.lax.bitcast_convert_type(
        a_tab.astype(jnp.bfloat16), jnp.uint16).astype(jnp.uint32)
    b_bits = jax.lax.bitcast_convert_type(
        b_tab.astype(jnp.bfloat16), jnp.uint16).astype(jnp.uint32)
    ab_packed = (a_bits | (b_bits << 16)).reshape(C * D)
    sc_apply = _make_sc_apply(N, D, C, rows_per_chunk=16)
    return sc_apply(samples, contexts.reshape(N), ab_packed)


# speedup vs baseline: 1.1665x; 1.1665x over previous
import functools

import jax
import jax.numpy as jnp
from jax import lax
from jax.experimental import pallas as pl
from jax.experimental.pallas import tpu as pltpu
from jax.experimental.pallas import tpu_sc as plsc

EPS = 0.001

_NC = 2
_NS = 16
_L = 16


def _fold_params_kernel(g_ref, b_ref, m_ref, v_ref, p_ref, a_out, b_out):
    inv = jax.lax.rsqrt(v_ref[...] + EPS) * g_ref[...]
    rp = jax.lax.rsqrt(p_ref[...])
    a_out[...] = inv * rp
    b_out[...] = (b_ref[...] - m_ref[...] * inv) * rp


def _make_sc_apply(N, D, C, rows_per_chunk):
    nw = _NC * _NS
    rows_per_w = N // nw
    n_chunks = rows_per_w // rows_per_chunk
    mesh = plsc.VectorSubcoreMesh(core_axis_name="c", subcore_axis_name="s")

    @functools.partial(
        pl.kernel,
        mesh=mesh,
        out_type=jax.ShapeDtypeStruct((N, D), jnp.float32),
        scratch_types=[
            pltpu.VMEM((rows_per_chunk, D), jnp.float32),
            pltpu.VMEM((rows_per_chunk, D), jnp.float32),
            pltpu.VMEM((rows_per_chunk, D), jnp.float32),
            pltpu.VMEM((rows_per_chunk, D), jnp.float32),
            pltpu.VMEM((rows_per_chunk, D), jnp.float32),
            pltpu.VMEM((rows_per_chunk, D), jnp.float32),
            pltpu.VMEM((rows_per_w,), jnp.int32),
            pltpu.VMEM((C * D,), jnp.uint32),
            pltpu.SemaphoreType.DMA,
            pltpu.SemaphoreType.DMA,
            pltpu.SemaphoreType.DMA,
            pltpu.SemaphoreType.DMA,
            pltpu.SemaphoreType.DMA,
            pltpu.SemaphoreType.DMA,
        ],
    )
    def sc_apply(x_hbm, ctx_hbm, ab_hbm, out_hbm,
                 xbuf0, xbuf1, xbuf2, obuf0, obuf1, obuf2, ctxbuf, abbuf,
                 isem0, isem1, isem2, osem0, osem1, osem2):
        wid = lax.axis_index("s") * _NC + lax.axis_index("c")
        base = wid * rows_per_w
        pltpu.sync_copy(ab_hbm, abbuf)
        pltpu.sync_copy(ctx_hbm.at[pl.ds(base, rows_per_w)], ctxbuf)

        xbufs, obufs = (xbuf0, xbuf1, xbuf2), (obuf0, obuf1, obuf2)
        isems, osems = (isem0, isem1, isem2), (osem0, osem1, osem2)
        unroll = 4
        row_group = 4
        depth = 3
        n_groups = n_chunks // depth

        def start_in(i, slot):
            rb = base + i * rows_per_chunk
            pltpu.async_copy(x_hbm.at[pl.ds(rb, rows_per_chunk)],
                             xbufs[slot], isems[slot])

        def wait_in(i, slot):
            rb = base + i * rows_per_chunk
            pltpu.make_async_copy(x_hbm.at[pl.ds(rb, rows_per_chunk)],
                                  xbufs[slot], isems[slot]).wait()

        def start_out(i, slot):
            rb = base + i * rows_per_chunk
            pltpu.async_copy(obufs[slot],
                             out_hbm.at[pl.ds(rb, rows_per_chunk)], osems[slot])

        def wait_out(i, slot):
            rb = base + i * rows_per_chunk
            pltpu.make_async_copy(obufs[slot],
                                  out_hbm.at[pl.ds(rb, rows_per_chunk)],
                                  osems[slot]).wait()

        def compute(i, slot):
            xbuf, obuf = xbufs[slot], obufs[slot]
            cvec = ctxbuf[pl.ds(i * rows_per_chunk, _L)]
            cbase = cvec * D
            cbs = [cbase[r] for r in range(rows_per_chunk)]
            for r0 in range(0, rows_per_chunk, row_group):

                @plsc.parallel_loop(0, D, _L, unroll=unroll)
                def col_body(off, r0=r0):
                    sl = pl.ds(off, _L)
                    for r in range(r0, r0 + row_group):
                        w = abbuf[pl.ds(cbs[r] + off, _L)]
                        a = jax.lax.bitcast_convert_type(w << 16, jnp.float32)
                        b = jax.lax.bitcast_convert_type(
                            w & jnp.uint32(0xFFFF0000), jnp.float32)
                        obuf[r, sl] = xbuf[r, sl] * a + b

        for s in range(depth):
            start_in(s, s)

        def group_body(g, _):
            for s in range(depth):
                i = depth * g + s
                wait_in(i, s)

                @pl.when(g > 0)
                def _():
                    wait_out(i - depth, s)

                compute(i, s)
                start_out(i, s)

                @pl.when(i + depth < n_chunks)
                def _():
                    start_in(i + depth, s)

            return 0

        lax.fori_loop(0, n_groups, group_body, 0)
        for i in range(depth * n_groups, n_chunks):
            s = i % depth
            wait_in(i, s)
            wait_out(i - depth, s)
            compute(i, s)
            start_out(i, s)
        for i in range(n_chunks - depth, n_chunks):
            wait_out(i, i % depth)

    return sc_apply


def kernel(samples, contexts, gamma, beta, moving_mean, moving_var, priors):
    N, D = samples.shape
    C = gamma.shape[0]
    a_tab, b_tab = pl.pallas_call(
        _fold_params_kernel,
        out_shape=(jax.ShapeDtypeStruct((C, D), jnp.float32),
                   jax.ShapeDtypeStruct((C, D), jnp.float32)),
    )(gamma, beta, moving_mean, moving_var, priors.reshape(C, 1))

    a_bits = jax.lax.bitcast_convert_type(
        a_tab.astype(jnp.bfloat16), jnp.uint16).astype(jnp.uint32)
    b_bits = jax.lax.bitcast_convert_type(
        b_tab.astype(jnp.bfloat16), jnp.uint16).astype(jnp.uint32)
    ab_packed = (a_bits | (b_bits << 16)).reshape(C * D)
    sc_apply = _make_sc_apply(N, D, C, rows_per_chunk=16)
    return sc_apply(samples, contexts.reshape(N), ab_packed)


# speedup vs baseline: 1.2646x; 1.0841x over previous
import functools

import jax
import jax.numpy as jnp
from jax import lax
from jax.experimental import pallas as pl
from jax.experimental.pallas import tpu as pltpu
from jax.experimental.pallas import tpu_sc as plsc

EPS = 0.001

_NC = 2
_NS = 16
_L = 16


def _fold_params_kernel(g_ref, b_ref, m_ref, v_ref, p_ref, a_out, b_out):
    inv = jax.lax.rsqrt(v_ref[...] + EPS) * g_ref[...]
    rp = jax.lax.rsqrt(p_ref[...])
    a_out[...] = inv * rp
    b_out[...] = (b_ref[...] - m_ref[...] * inv) * rp


def _make_sc_apply(N, D, C, rows_per_chunk):
    nw = _NC * _NS
    rows_per_w = N // nw
    n_chunks = rows_per_w // rows_per_chunk
    mesh = plsc.VectorSubcoreMesh(core_axis_name="c", subcore_axis_name="s")

    @functools.partial(
        pl.kernel,
        mesh=mesh,
        out_type=jax.ShapeDtypeStruct((N, D), jnp.float32),
        scratch_types=[
            pltpu.VMEM((rows_per_chunk, D), jnp.float32),
            pltpu.VMEM((rows_per_chunk, D), jnp.float32),
            pltpu.VMEM((rows_per_chunk, D), jnp.float32),
            pltpu.VMEM((rows_per_chunk, D), jnp.float32),
            pltpu.VMEM((rows_per_chunk, D), jnp.float32),
            pltpu.VMEM((rows_per_chunk, D), jnp.float32),
            pltpu.VMEM((rows_per_w,), jnp.int32),
            pltpu.VMEM((C * D,), jnp.uint32),
            pltpu.SemaphoreType.DMA,
            pltpu.SemaphoreType.DMA,
            pltpu.SemaphoreType.DMA,
            pltpu.SemaphoreType.DMA,
            pltpu.SemaphoreType.DMA,
            pltpu.SemaphoreType.DMA,
        ],
    )
    def sc_apply(x_hbm, ctx_hbm, ab_hbm, out_hbm,
                 xbuf0, xbuf1, xbuf2, obuf0, obuf1, obuf2, ctxbuf, abbuf,
                 isem0, isem1, isem2, osem0, osem1, osem2):
        wid = lax.axis_index("s") * _NC + lax.axis_index("c")
        base = wid * rows_per_w
        pltpu.sync_copy(ab_hbm, abbuf)
        pltpu.sync_copy(ctx_hbm.at[pl.ds(base, rows_per_w)], ctxbuf)

        xbufs, obufs = (xbuf0, xbuf1, xbuf2), (obuf0, obuf1, obuf2)
        isems, osems = (isem0, isem1, isem2), (osem0, osem1, osem2)
        unroll = 8
        row_group = 1
        depth = 3
        n_groups = n_chunks // depth

        def start_in(i, slot):
            rb = base + i * rows_per_chunk
            pltpu.async_copy(x_hbm.at[pl.ds(rb, rows_per_chunk)],
                             xbufs[slot], isems[slot])

        def wait_in(i, slot):
            rb = base + i * rows_per_chunk
            pltpu.make_async_copy(x_hbm.at[pl.ds(rb, rows_per_chunk)],
                                  xbufs[slot], isems[slot]).wait()

        def start_out(i, slot):
            rb = base + i * rows_per_chunk
            pltpu.async_copy(obufs[slot],
                             out_hbm.at[pl.ds(rb, rows_per_chunk)], osems[slot])

        def wait_out(i, slot):
            rb = base + i * rows_per_chunk
            pltpu.make_async_copy(obufs[slot],
                                  out_hbm.at[pl.ds(rb, rows_per_chunk)],
                                  osems[slot]).wait()

        def compute(i, slot):
            xbuf, obuf = xbufs[slot], obufs[slot]
            cvec = ctxbuf[pl.ds(i * rows_per_chunk, _L)]
            cbase = cvec * D
            cbs = [cbase[r] for r in range(rows_per_chunk)]
            for r0 in range(0, rows_per_chunk, row_group):

                @plsc.parallel_loop(0, D, _L, unroll=unroll)
                def col_body(off, r0=r0):
                    sl = pl.ds(off, _L)
                    for r in range(r0, r0 + row_group):
                        w = abbuf[pl.ds(cbs[r] + off, _L)]
                        a = jax.lax.bitcast_convert_type(w << 16, jnp.float32)
                        b = jax.lax.bitcast_convert_type(
                            w & jnp.uint32(0xFFFF0000), jnp.float32)
                        obuf[r, sl] = xbuf[r, sl] * a + b

        for s in range(depth):
            start_in(s, s)

        def group_body(g, _):
            for s in range(depth):
                i = depth * g + s
                wait_in(i, s)

                @pl.when(g > 0)
                def _():
                    wait_out(i - depth, s)

                compute(i, s)
                start_out(i, s)

                @pl.when(i + depth < n_chunks)
                def _():
                    start_in(i + depth, s)

            return 0

        lax.fori_loop(0, n_groups, group_body, 0)
        for i in range(depth * n_groups, n_chunks):
            s = i % depth
            wait_in(i, s)
            wait_out(i - depth, s)
            compute(i, s)
            start_out(i, s)
        for i in range(n_chunks - depth, n_chunks):
            wait_out(i, i % depth)

    return sc_apply


def kernel(samples, contexts, gamma, beta, moving_mean, moving_var, priors):
    N, D = samples.shape
    C = gamma.shape[0]
    a_tab, b_tab = pl.pallas_call(
        _fold_params_kernel,
        out_shape=(jax.ShapeDtypeStruct((C, D), jnp.float32),
                   jax.ShapeDtypeStruct((C, D), jnp.float32)),
    )(gamma, beta, moving_mean, moving_var, priors.reshape(C, 1))

    a_bits = jax.lax.bitcast_convert_type(
        a_tab.astype(jnp.bfloat16), jnp.uint16).astype(jnp.uint32)
    b_bits = jax.lax.bitcast_convert_type(
        b_tab.astype(jnp.bfloat16), jnp.uint16).astype(jnp.uint32)
    ab_packed = (a_bits | (b_bits << 16)).reshape(C * D)
    sc_apply = _make_sc_apply(N, D, C, rows_per_chunk=16)
    return sc_apply(samples, contexts.reshape(N), ab_packed)
